# Initial kernel scaffold; baseline (speedup 1.0000x reference)
import jax
import jax.numpy as jnp
from jax.experimental import pallas as pl


def kernel(x, edge_index, W1, b1, W2, b2):
    raise NotImplementedError("write your pallas kernel here")


# speedup vs baseline: 18.9168x; 18.9168x over previous
import functools

import jax
import jax.numpy as jnp
from jax import lax
from jax.experimental import pallas as pl
from jax.experimental.pallas import tpu as pltpu
from jax.experimental.pallas import tpu_sc as plsc

N_NODES = 100000
N_PAD = 102400
N_EDGES = 1600000
F_IN = 16
F_HALF = 8
H_MID = 32
NC, NS = 2, 16
NW = NC * NS
CHUNK = 125
DKCH = N_EDGES // (NW * CHUNK)
AKCH = N_EDGES // (NS * CHUNK)
ABK = 80
ANB = AKCH // ABK
ROWS_PER_TILE = N_PAD // NS


def _deg_body(dst_hbm, zeros_hbm, ones_hbm, pdeg_hbm, dstv, onesv, deg_sp):
  cid = lax.axis_index("c")
  sid = lax.axis_index("s")
  wid = sid * NC + cid
  r0 = sid * ROWS_PER_TILE
  pltpu.sync_copy(zeros_hbm.at[pl.ds(r0, ROWS_PER_TILE)],
                  deg_sp.at[pl.ds(r0, ROWS_PER_TILE)])
  pltpu.sync_copy(dst_hbm.at[wid], dstv)
  pltpu.sync_copy(ones_hbm, onesv)
  plsc.subcore_barrier()

  def body(j, carry):
    pltpu.sync_copy(onesv, deg_sp.at[dstv.at[j]], add=True)
    return carry

  lax.fori_loop(0, DKCH, body, 0)
  plsc.subcore_barrier()
  pltpu.sync_copy(deg_sp.at[pl.ds(r0, ROWS_PER_TILE)],
                  pdeg_hbm.at[pl.ds(cid * N_PAD + r0, ROWS_PER_TILE)])


def _agg_body(src_hbm, dst_hbm, table_hbm, zeros_hbm, out_hbm,
              srcv, dstv, rows, sem, agg_sp):
  cid = lax.axis_index("c")
  sid = lax.axis_index("s")
  r0 = sid * ROWS_PER_TILE
  pltpu.sync_copy(zeros_hbm.at[pl.ds(r0, ROWS_PER_TILE)],
                  agg_sp.at[pl.ds(r0, ROWS_PER_TILE)])
  plsc.subcore_barrier()

  def blk_body(b, carry):
    pltpu.sync_copy(src_hbm.at[cid, sid, pl.ds(b * ABK, ABK)], srcv)
    pltpu.sync_copy(dst_hbm.at[sid, pl.ds(b * ABK, ABK)], dstv)

    def body(j, c2):
      pltpu.async_copy(table_hbm.at[srcv.at[j]], rows, sem).wait()
      pltpu.sync_copy(rows, agg_sp.at[dstv.at[j]], add=True)
      return c2

    lax.fori_loop(0, ABK, body, 0)
    return carry

  lax.fori_loop(0, ANB, blk_body, 0)
  plsc.subcore_barrier()
  pltpu.sync_copy(agg_sp.at[pl.ds(r0, ROWS_PER_TILE)],
                  out_hbm.at[cid, pl.ds(r0, ROWS_PER_TILE)])


@functools.cache
def _sc_calls():
  mesh = plsc.VectorSubcoreMesh(
      core_axis_name="c", subcore_axis_name="s",
      num_cores=NC, num_subcores=NS)
  params = pltpu.CompilerParams(use_tc_tiling_on_sc=False)
  deg_call = pl.kernel(
      _deg_body,
      out_type=jax.ShapeDtypeStruct((NC * N_PAD,), jnp.float32),
      mesh=mesh,
      compiler_params=params,
      scratch_types=[
          pltpu.VMEM((DKCH, CHUNK), jnp.int32),
          pltpu.VMEM((CHUNK,), jnp.float32),
          pltpu.VMEM_SHARED((N_PAD,), jnp.float32),
      ],
  )
  agg_call = pl.kernel(
      _agg_body,
      out_type=jax.ShapeDtypeStruct((NC, N_PAD, F_HALF), jnp.float32),
      mesh=mesh,
      compiler_params=params,
      scratch_types=[
          pltpu.VMEM((ABK, CHUNK), jnp.int32),
          pltpu.VMEM((ABK, CHUNK), jnp.int32),
          pltpu.VMEM((CHUNK, F_HALF), jnp.float32),
          pltpu.SemaphoreType.DMA,
          pltpu.VMEM_SHARED((N_PAD, F_HALF), jnp.float32),
      ],
  )
  return deg_call, agg_call


_BLK = 6400
_GRID = N_PAD // _BLK
_EROWS = 1280
_ECOLS = 1250
_EBLK = _EROWS // _GRID


def _prep_body(pdeg_ref, x_ref, src_ref, xnb_ref, dis_ref, srcs_ref):
  deg = pdeg_ref[:, 0:1] + pdeg_ref[:, 1:2] + 1.0
  dis = lax.rsqrt(deg)
  dis_ref[...] = dis
  xnb_ref[0] = x_ref[:, :F_HALF] * dis
  xnb_ref[1] = x_ref[:, F_HALF:] * dis
  srcs_ref[0] = src_ref[...]
  srcs_ref[1] = src_ref[...] + N_PAD


_prep_call = pl.pallas_call(
    _prep_body,
    grid=(_GRID,),
    in_specs=[
        pl.BlockSpec((_BLK, NC), lambda i: (i, 0)),
        pl.BlockSpec((_BLK, F_IN), lambda i: (i, 0)),
        pl.BlockSpec((_EBLK, _ECOLS), lambda i: (i, 0)),
    ],
    out_specs=[
        pl.BlockSpec((NC, _BLK, F_HALF), lambda i: (0, i, 0)),
        pl.BlockSpec((_BLK, 1), lambda i: (i, 0)),
        pl.BlockSpec((NC, _EBLK, _ECOLS), lambda i: (0, i, 0)),
    ],
    out_shape=[
        jax.ShapeDtypeStruct((NC, N_PAD, F_HALF), jnp.float32),
        jax.ShapeDtypeStruct((N_PAD, 1), jnp.float32),
        jax.ShapeDtypeStruct((NC, _EROWS, _ECOLS), jnp.int32),
    ],
)


def _mid_body(agg_ref, x_ref, dis_ref, w1_ref, b1_ref, w2_ref, gn_ref):
  dis = dis_ref[...]
  agg16 = jnp.concatenate([agg_ref[0], agg_ref[1]], axis=1)
  s = agg16 * dis + x_ref[...] * (dis * dis)
  h1 = jnp.dot(s, w1_ref[...], preferred_element_type=jnp.float32)
  h1 = jnp.maximum(h1 + b1_ref[...], 0.0)
  g = jnp.dot(h1, w2_ref[...], preferred_element_type=jnp.float32)
  gn_ref[0] = g[:, :F_HALF] * dis
  gn_ref[1] = g[:, F_HALF:] * dis


_mid_call = pl.pallas_call(
    _mid_body,
    grid=(_GRID,),
    in_specs=[
        pl.BlockSpec((NC, _BLK, F_HALF), lambda i: (0, i, 0)),
        pl.BlockSpec((_BLK, F_IN), lambda i: (i, 0)),
        pl.BlockSpec((_BLK, 1), lambda i: (i, 0)),
        pl.BlockSpec((F_IN, H_MID), lambda i: (0, 0)),
        pl.BlockSpec((1, H_MID), lambda i: (0, 0)),
        pl.BlockSpec((H_MID, F_IN), lambda i: (0, 0)),
    ],
    out_specs=pl.BlockSpec((NC, _BLK, F_HALF), lambda i: (0, i, 0)),
    out_shape=jax.ShapeDtypeStruct((NC, N_PAD, F_HALF), jnp.float32),
)


def _fin_body(agg_ref, gn_ref, dis_ref, b2_ref, x_ref, out_ref):
  agg16 = jnp.concatenate([agg_ref[0], agg_ref[1]], axis=1)
  gn16 = jnp.concatenate([gn_ref[0], gn_ref[1]], axis=1)
  t = (agg16 + gn16) * dis_ref[...] + b2_ref[...]
  out_ref[...] = 0.2 * t + 0.8 * x_ref[...]


_fin_call = pl.pallas_call(
    _fin_body,
    grid=(_GRID,),
    in_specs=[
        pl.BlockSpec((NC, _BLK, F_HALF), lambda i: (0, i, 0)),
        pl.BlockSpec((NC, _BLK, F_HALF), lambda i: (0, i, 0)),
        pl.BlockSpec((_BLK, 1), lambda i: (i, 0)),
        pl.BlockSpec((1, F_IN), lambda i: (0, 0)),
        pl.BlockSpec((_BLK, F_IN), lambda i: (i, 0)),
    ],
    out_specs=pl.BlockSpec((_BLK, F_IN), lambda i: (i, 0)),
    out_shape=jax.ShapeDtypeStruct((N_PAD, F_IN), jnp.float32),
)


def kernel(x, edge_index, W1, b1, W2, b2):
  ei = edge_index.astype(jnp.int32)
  src2d = ei[0].reshape(_EROWS, _ECOLS)
  dst_deg = ei[1].reshape(NW, DKCH, CHUNK)
  dst_agg = ei[1].reshape(NS, AKCH, CHUNK)
  xp = jnp.pad(x, ((0, N_PAD - N_NODES), (0, 0)))
  z1 = jnp.zeros((N_PAD,), jnp.float32)
  z8 = jnp.zeros((N_PAD, F_HALF), jnp.float32)
  ones = jnp.ones((CHUNK,), jnp.float32)

  _deg_call, _agg_call = _sc_calls()
  pdeg = _deg_call(dst_deg, z1, ones).reshape(NC, N_PAD)
  xnb, dis, srcs = _prep_call(pdeg.T, xp, src2d)
  srcs4 = srcs.reshape(NC, NS, AKCH, CHUNK)
  xnb_t = xnb.reshape(NC * N_PAD, F_HALF)
  agg1 = _agg_call(srcs4, dst_agg, xnb_t, z8)
  gn = _mid_call(agg1, xp, dis, W1, b1.reshape(1, H_MID), W2)
  gn_t = gn.reshape(NC * N_PAD, F_HALF)
  agg2 = _agg_call(srcs4, dst_agg, gn_t, z8)
  outp = _fin_call(agg2, gn, dis, b2.reshape(1, F_IN), xp)
  return outp[:N_NODES]


# speedup vs baseline: 33.8102x; 1.7873x over previous
import functools

import jax
import jax.numpy as jnp
from jax import lax
from jax.experimental import pallas as pl
from jax.experimental.pallas import tpu as pltpu
from jax.experimental.pallas import tpu_sc as plsc

N_NODES = 100000
N_PAD = 102400
N_EDGES = 1600000
F_IN = 16
F_HALF = 8
H_MID = 32
NC, NS = 2, 16
NW = NC * NS
CHUNK = 125
DKCH = N_EDGES // (NW * CHUNK)
AKCH = N_EDGES // (NS * CHUNK)
ABK = 80
ANB = AKCH // ABK
KBUF = 8
ROWS_PER_TILE = N_PAD // NS


def _deg_body(dst_hbm, zeros_hbm, ones_hbm, pdeg_hbm, dstv, onesv, deg_sp):
  cid = lax.axis_index("c")
  sid = lax.axis_index("s")
  wid = sid * NC + cid
  r0 = sid * ROWS_PER_TILE
  pltpu.sync_copy(zeros_hbm.at[pl.ds(r0, ROWS_PER_TILE)],
                  deg_sp.at[pl.ds(r0, ROWS_PER_TILE)])
  pltpu.sync_copy(dst_hbm.at[wid], dstv)
  pltpu.sync_copy(ones_hbm, onesv)
  plsc.subcore_barrier()

  def body(j, carry):
    pltpu.sync_copy(onesv, deg_sp.at[dstv.at[j]], add=True)
    return carry

  lax.fori_loop(0, DKCH, body, 0)
  plsc.subcore_barrier()
  pltpu.sync_copy(deg_sp.at[pl.ds(r0, ROWS_PER_TILE)],
                  pdeg_hbm.at[pl.ds(cid * N_PAD + r0, ROWS_PER_TILE)])


def _agg_body(src_hbm, dst_hbm, table_hbm, zeros_hbm, out_hbm,
              srcv, dstv, rows, sem, agg_sp):
  cid = lax.axis_index("c")
  sid = lax.axis_index("s")
  r0 = sid * ROWS_PER_TILE
  pltpu.sync_copy(zeros_hbm.at[pl.ds(r0, ROWS_PER_TILE)],
                  agg_sp.at[pl.ds(r0, ROWS_PER_TILE)])
  plsc.subcore_barrier()

  def blk_body(b, carry):
    pltpu.sync_copy(src_hbm.at[cid, sid, pl.ds(b * ABK, ABK)], srcv)
    pltpu.sync_copy(dst_hbm.at[sid, pl.ds(b * ABK, ABK)], dstv)

    def group(g, c2):
      j0 = g * KBUF
      copies = [
          pltpu.async_copy(table_hbm.at[srcv.at[j0 + k]], rows.at[k],
                           sem.at[k])
          for k in range(KBUF)
      ]
      for k in range(KBUF):
        copies[k].wait()
        pltpu.sync_copy(rows.at[k], agg_sp.at[dstv.at[j0 + k]], add=True)
      return c2

    lax.fori_loop(0, ABK // KBUF, group, 0)
    return carry

  lax.fori_loop(0, ANB, blk_body, 0)
  plsc.subcore_barrier()
  pltpu.sync_copy(agg_sp.at[pl.ds(r0, ROWS_PER_TILE)],
                  out_hbm.at[cid, pl.ds(r0, ROWS_PER_TILE)])


@functools.cache
def _sc_calls():
  mesh = plsc.VectorSubcoreMesh(
      core_axis_name="c", subcore_axis_name="s",
      num_cores=NC, num_subcores=NS)
  params = pltpu.CompilerParams(use_tc_tiling_on_sc=False)
  deg_call = pl.kernel(
      _deg_body,
      out_type=jax.ShapeDtypeStruct((NC * N_PAD,), jnp.float32),
      mesh=mesh,
      compiler_params=params,
      scratch_types=[
          pltpu.VMEM((DKCH, CHUNK), jnp.int32),
          pltpu.VMEM((CHUNK,), jnp.float32),
          pltpu.VMEM_SHARED((N_PAD,), jnp.float32),
      ],
  )
  agg_call = pl.kernel(
      _agg_body,
      out_type=jax.ShapeDtypeStruct((NC, N_PAD, F_HALF), jnp.float32),
      mesh=mesh,
      compiler_params=params,
      scratch_types=[
          pltpu.VMEM((ABK, CHUNK), jnp.int32),
          pltpu.VMEM((ABK, CHUNK), jnp.int32),
          pltpu.VMEM((KBUF, CHUNK, F_HALF), jnp.float32),
          pltpu.SemaphoreType.DMA((KBUF,)),
          pltpu.VMEM_SHARED((N_PAD, F_HALF), jnp.float32),
      ],
  )
  return deg_call, agg_call


_BLK = 6400
_GRID = N_PAD // _BLK
_EROWS = 1280
_ECOLS = 1250
_EBLK = _EROWS // _GRID


def _prep_body(pdeg_ref, x_ref, src_ref, xnb_ref, dis_ref, srcs_ref):
  deg = pdeg_ref[:, 0:1] + pdeg_ref[:, 1:2] + 1.0
  dis = lax.rsqrt(deg)
  dis_ref[...] = dis
  xnb_ref[0] = x_ref[:, :F_HALF] * dis
  xnb_ref[1] = x_ref[:, F_HALF:] * dis
  srcs_ref[0] = src_ref[...]
  srcs_ref[1] = src_ref[...] + N_PAD


_prep_call = pl.pallas_call(
    _prep_body,
    grid=(_GRID,),
    in_specs=[
        pl.BlockSpec((_BLK, NC), lambda i: (i, 0)),
        pl.BlockSpec((_BLK, F_IN), lambda i: (i, 0)),
        pl.BlockSpec((_EBLK, _ECOLS), lambda i: (i, 0)),
    ],
    out_specs=[
        pl.BlockSpec((NC, _BLK, F_HALF), lambda i: (0, i, 0)),
        pl.BlockSpec((_BLK, 1), lambda i: (i, 0)),
        pl.BlockSpec((NC, _EBLK, _ECOLS), lambda i: (0, i, 0)),
    ],
    out_shape=[
        jax.ShapeDtypeStruct((NC, N_PAD, F_HALF), jnp.float32),
        jax.ShapeDtypeStruct((N_PAD, 1), jnp.float32),
        jax.ShapeDtypeStruct((NC, _EROWS, _ECOLS), jnp.int32),
    ],
)


def _mid_body(agg_ref, x_ref, dis_ref, w1_ref, b1_ref, w2_ref, gn_ref):
  dis = dis_ref[...]
  agg16 = jnp.concatenate([agg_ref[0], agg_ref[1]], axis=1)
  s = agg16 * dis + x_ref[...] * (dis * dis)
  h1 = jnp.dot(s, w1_ref[...], preferred_element_type=jnp.float32)
  h1 = jnp.maximum(h1 + b1_ref[...], 0.0)
  g = jnp.dot(h1, w2_ref[...], preferred_element_type=jnp.float32)
  gn_ref[0] = g[:, :F_HALF] * dis
  gn_ref[1] = g[:, F_HALF:] * dis


_mid_call = pl.pallas_call(
    _mid_body,
    grid=(_GRID,),
    in_specs=[
        pl.BlockSpec((NC, _BLK, F_HALF), lambda i: (0, i, 0)),
        pl.BlockSpec((_BLK, F_IN), lambda i: (i, 0)),
        pl.BlockSpec((_BLK, 1), lambda i: (i, 0)),
        pl.BlockSpec((F_IN, H_MID), lambda i: (0, 0)),
        pl.BlockSpec((1, H_MID), lambda i: (0, 0)),
        pl.BlockSpec((H_MID, F_IN), lambda i: (0, 0)),
    ],
    out_specs=pl.BlockSpec((NC, _BLK, F_HALF), lambda i: (0, i, 0)),
    out_shape=jax.ShapeDtypeStruct((NC, N_PAD, F_HALF), jnp.float32),
)


def _fin_body(agg_ref, gn_ref, dis_ref, b2_ref, x_ref, out_ref):
  agg16 = jnp.concatenate([agg_ref[0], agg_ref[1]], axis=1)
  gn16 = jnp.concatenate([gn_ref[0], gn_ref[1]], axis=1)
  t = (agg16 + gn16) * dis_ref[...] + b2_ref[...]
  out_ref[...] = 0.2 * t + 0.8 * x_ref[...]


_fin_call = pl.pallas_call(
    _fin_body,
    grid=(_GRID,),
    in_specs=[
        pl.BlockSpec((NC, _BLK, F_HALF), lambda i: (0, i, 0)),
        pl.BlockSpec((NC, _BLK, F_HALF), lambda i: (0, i, 0)),
        pl.BlockSpec((_BLK, 1), lambda i: (i, 0)),
        pl.BlockSpec((1, F_IN), lambda i: (0, 0)),
        pl.BlockSpec((_BLK, F_IN), lambda i: (i, 0)),
    ],
    out_specs=pl.BlockSpec((_BLK, F_IN), lambda i: (i, 0)),
    out_shape=jax.ShapeDtypeStruct((N_PAD, F_IN), jnp.float32),
)


def kernel(x, edge_index, W1, b1, W2, b2):
  ei = edge_index.astype(jnp.int32)
  src2d = ei[0].reshape(_EROWS, _ECOLS)
  dst_deg = ei[1].reshape(NW, DKCH, CHUNK)
  dst_agg = ei[1].reshape(NS, AKCH, CHUNK)
  xp = jnp.pad(x, ((0, N_PAD - N_NODES), (0, 0)))
  z1 = jnp.zeros((N_PAD,), jnp.float32)
  z8 = jnp.zeros((N_PAD, F_HALF), jnp.float32)
  ones = jnp.ones((CHUNK,), jnp.float32)

  _deg_call, _agg_call = _sc_calls()
  pdeg = _deg_call(dst_deg, z1, ones).reshape(NC, N_PAD)
  xnb, dis, srcs = _prep_call(pdeg.T, xp, src2d)
  srcs4 = srcs.reshape(NC, NS, AKCH, CHUNK)
  xnb_t = xnb.reshape(NC * N_PAD, F_HALF)
  agg1 = _agg_call(srcs4, dst_agg, xnb_t, z8)
  gn = _mid_call(agg1, xp, dis, W1, b1.reshape(1, H_MID), W2)
  gn_t = gn.reshape(NC * N_PAD, F_HALF)
  agg2 = _agg_call(srcs4, dst_agg, gn_t, z8)
  outp = _fin_call(agg2, gn, dis, b2.reshape(1, F_IN), xp)
  return outp[:N_NODES]


# speedup vs baseline: 35.5956x; 1.0528x over previous
import functools

import jax
import jax.numpy as jnp
from jax import lax
from jax.experimental import pallas as pl
from jax.experimental.pallas import tpu as pltpu
from jax.experimental.pallas import tpu_sc as plsc

N_NODES = 100000
N_PAD = 102400
N_EDGES = 1600000
F_IN = 16
F_HALF = 8
H_MID = 32
NC, NS = 2, 16
CHUNK = 128
E_PAD = 1605632
AKCH = E_PAD // (NS * CHUNK)
DKCH = AKCH // NC
ABK = 56
ANB = AKCH // ABK
KBUF = 8
ROWS_PER_TILE = N_PAD // NS


def _deg_body(dst_hbm, zeros_hbm, ones_hbm, pdeg_hbm, dstv, onesv, deg_sp):
  cid = lax.axis_index("c")
  sid = lax.axis_index("s")
  r0 = sid * ROWS_PER_TILE
  pltpu.sync_copy(zeros_hbm.at[pl.ds(r0, ROWS_PER_TILE)],
                  deg_sp.at[pl.ds(r0, ROWS_PER_TILE)])
  pltpu.sync_copy(dst_hbm.at[sid, pl.ds(cid * DKCH, DKCH)], dstv)
  pltpu.sync_copy(ones_hbm, onesv)
  plsc.subcore_barrier()

  def body(j, carry):
    pltpu.sync_copy(onesv, deg_sp.at[dstv.at[j]], add=True)
    return carry

  lax.fori_loop(0, DKCH, body, 0)
  plsc.subcore_barrier()
  pltpu.sync_copy(deg_sp.at[pl.ds(r0, ROWS_PER_TILE)],
                  pdeg_hbm.at[pl.ds(cid * N_PAD + r0, ROWS_PER_TILE)])


def _agg_body(src_hbm, dst_hbm, tlo_hbm, thi_hbm, zeros_hbm, out_hbm,
              srcv, dstv, rows, gsem, ssem, agg_sp):
  cid = lax.axis_index("c")
  sid = lax.axis_index("s")
  r0 = sid * ROWS_PER_TILE
  pltpu.sync_copy(zeros_hbm.at[pl.ds(r0, ROWS_PER_TILE)],
                  agg_sp.at[pl.ds(r0, ROWS_PER_TILE)])
  plsc.subcore_barrier()

  def run_all(table_hbm):
    def blk_body(b, carry):
      pltpu.sync_copy(src_hbm.at[sid, pl.ds(b * ABK, ABK)], srcv)
      pltpu.sync_copy(dst_hbm.at[sid, pl.ds(b * ABK, ABK)], dstv)

      def group(g, c2):
        j0 = g * KBUF
        gets = [
            pltpu.async_copy(table_hbm.at[srcv.at[j0 + k]], rows.at[k],
                             gsem.at[k])
            for k in range(KBUF)
        ]
        puts = []
        for k in range(KBUF):
          gets[k].wait()
          puts.append(
              pltpu.async_copy(rows.at[k], agg_sp.at[dstv.at[j0 + k]],
                               ssem.at[k], add=True))
        for p in puts:
          p.wait()
        return c2

      lax.fori_loop(0, ABK // KBUF, group, 0)
      return carry

    lax.fori_loop(0, ANB, blk_body, 0)

  @pl.when(cid == 0)
  def _():
    run_all(tlo_hbm)

  @pl.when(cid == 1)
  def _():
    run_all(thi_hbm)

  plsc.subcore_barrier()
  pltpu.sync_copy(agg_sp.at[pl.ds(r0, ROWS_PER_TILE)],
                  out_hbm.at[cid, pl.ds(r0, ROWS_PER_TILE)])


@functools.cache
def _sc_calls():
  mesh = plsc.VectorSubcoreMesh(
      core_axis_name="c", subcore_axis_name="s",
      num_cores=NC, num_subcores=NS)
  params = pltpu.CompilerParams(use_tc_tiling_on_sc=False)
  deg_call = pl.kernel(
      _deg_body,
      out_type=jax.ShapeDtypeStruct((NC * N_PAD,), jnp.float32),
      mesh=mesh,
      compiler_params=params,
      scratch_types=[
          pltpu.VMEM((DKCH, CHUNK), jnp.int32),
          pltpu.VMEM((CHUNK,), jnp.float32),
          pltpu.VMEM_SHARED((N_PAD,), jnp.float32),
      ],
  )
  agg_call = pl.kernel(
      _agg_body,
      out_type=jax.ShapeDtypeStruct((NC, N_PAD, F_HALF), jnp.float32),
      mesh=mesh,
      compiler_params=params,
      scratch_types=[
          pltpu.VMEM((ABK, CHUNK), jnp.int32),
          pltpu.VMEM((ABK, CHUNK), jnp.int32),
          pltpu.VMEM((KBUF, CHUNK, F_HALF), jnp.float32),
          pltpu.SemaphoreType.DMA((KBUF,)),
          pltpu.SemaphoreType.DMA((KBUF,)),
          pltpu.VMEM_SHARED((N_PAD, F_HALF), jnp.float32),
      ],
  )
  return deg_call, agg_call


_BLK = 2560
_GRID = N_PAD // _BLK


def _prep_body(pdeg_ref, x_ref, dis_ref, tlo_ref, thi_ref):
  deg = pdeg_ref[:, 0:1] + pdeg_ref[:, 1:2] + 1.0
  dis = lax.rsqrt(deg)
  dis_ref[...] = dis
  xn = x_ref[...] * dis
  tlo_ref[...] = xn[:, :F_HALF]
  thi_ref[...] = xn[:, F_HALF:]


_prep_call = pl.pallas_call(
    _prep_body,
    grid=(_GRID,),
    in_specs=[
        pl.BlockSpec((_BLK, NC), lambda i: (i, 0)),
        pl.BlockSpec((_BLK, F_IN), lambda i: (i, 0)),
    ],
    out_specs=[
        pl.BlockSpec((_BLK, 1), lambda i: (i, 0)),
        pl.BlockSpec((_BLK, F_HALF), lambda i: (i, 0)),
        pl.BlockSpec((_BLK, F_HALF), lambda i: (i, 0)),
    ],
    out_shape=[
        jax.ShapeDtypeStruct((N_PAD, 1), jnp.float32),
        jax.ShapeDtypeStruct((N_PAD, F_HALF), jnp.float32),
        jax.ShapeDtypeStruct((N_PAD, F_HALF), jnp.float32),
    ],
)


def _mid_body(agg_ref, dis_ref, x_ref, w1_ref, b1_ref, w2_ref,
              gn16_ref, glo_ref, ghi_ref):
  a16 = jnp.concatenate([agg_ref[0], agg_ref[1]], axis=1)
  dis = dis_ref[...]
  s = a16 * dis + x_ref[...] * (dis * dis)
  h1 = jnp.dot(s, w1_ref[...], preferred_element_type=jnp.float32)
  h1 = jnp.maximum(h1 + b1_ref[...], 0.0)
  g = jnp.dot(h1, w2_ref[...], preferred_element_type=jnp.float32)
  gn = g * dis
  gn16_ref[...] = gn
  glo_ref[...] = gn[:, :F_HALF]
  ghi_ref[...] = gn[:, F_HALF:]


_mid_call = pl.pallas_call(
    _mid_body,
    grid=(_GRID,),
    in_specs=[
        pl.BlockSpec((NC, _BLK, F_HALF), lambda i: (0, i, 0)),
        pl.BlockSpec((_BLK, 1), lambda i: (i, 0)),
        pl.BlockSpec((_BLK, F_IN), lambda i: (i, 0)),
        pl.BlockSpec((F_IN, H_MID), lambda i: (0, 0)),
        pl.BlockSpec((1, H_MID), lambda i: (0, 0)),
        pl.BlockSpec((H_MID, F_IN), lambda i: (0, 0)),
    ],
    out_specs=[
        pl.BlockSpec((_BLK, F_IN), lambda i: (i, 0)),
        pl.BlockSpec((_BLK, F_HALF), lambda i: (i, 0)),
        pl.BlockSpec((_BLK, F_HALF), lambda i: (i, 0)),
    ],
    out_shape=[
        jax.ShapeDtypeStruct((N_PAD, F_IN), jnp.float32),
        jax.ShapeDtypeStruct((N_PAD, F_HALF), jnp.float32),
        jax.ShapeDtypeStruct((N_PAD, F_HALF), jnp.float32),
    ],
)


def _fin_body(agg_ref, gn16_ref, dis_ref, b2_ref, x_ref, out_ref):
  a16 = jnp.concatenate([agg_ref[0], agg_ref[1]], axis=1)
  t = (a16 + gn16_ref[...]) * dis_ref[...] + b2_ref[...]
  out_ref[...] = 0.2 * t + 0.8 * x_ref[...]


_fin_call = pl.pallas_call(
    _fin_body,
    grid=(_GRID,),
    in_specs=[
        pl.BlockSpec((NC, _BLK, F_HALF), lambda i: (0, i, 0)),
        pl.BlockSpec((_BLK, F_IN), lambda i: (i, 0)),
        pl.BlockSpec((_BLK, 1), lambda i: (i, 0)),
        pl.BlockSpec((1, F_IN), lambda i: (0, 0)),
        pl.BlockSpec((_BLK, F_IN), lambda i: (i, 0)),
    ],
    out_specs=pl.BlockSpec((_BLK, F_IN), lambda i: (i, 0)),
    out_shape=jax.ShapeDtypeStruct((N_NODES, F_IN), jnp.float32),
)


def kernel(x, edge_index, W1, b1, W2, b2):
  ei = edge_index.astype(jnp.int32)
  pad_n = E_PAD - N_EDGES
  srcp = jnp.pad(ei[0], (0, pad_n), constant_values=N_NODES).reshape(
      NS, AKCH, CHUNK)
  dstp = jnp.pad(ei[1], (0, pad_n), constant_values=N_NODES).reshape(
      NS, AKCH, CHUNK)
  xp = jnp.pad(x, ((0, N_PAD - N_NODES), (0, 0)))
  z1 = jnp.zeros((N_PAD,), jnp.float32)
  z8 = jnp.zeros((N_PAD, F_HALF), jnp.float32)
  ones = jnp.ones((CHUNK,), jnp.float32)

  _deg_call, _agg_call = _sc_calls()
  pdeg = _deg_call(dstp, z1, ones)
  pdegT = pdeg.reshape(NC, N_PAD).T
  dis, tlo, thi = _prep_call(pdegT, xp)
  agg1 = _agg_call(srcp, dstp, tlo, thi, z8)
  gn16, glo, ghi = _mid_call(agg1, dis, xp, W1, b1.reshape(1, H_MID), W2)
  agg2 = _agg_call(srcp, dstp, glo, ghi, z8)
  return _fin_call(agg2, gn16, dis, b2.reshape(1, F_IN), xp)


# speedup vs baseline: 37.4674x; 1.0526x over previous
import functools

import jax
import jax.numpy as jnp
from jax import lax
from jax.experimental import pallas as pl
from jax.experimental.pallas import tpu as pltpu
from jax.experimental.pallas import tpu_sc as plsc

N_NODES = 100000
N_PAD = 102400
N_EDGES = 1600000
F_IN = 16
F_HALF = 8
H_MID = 32
NC, NS = 2, 16
CHUNK = 128
E_PAD = 1605632
AKCH = E_PAD // (NS * CHUNK)
DKCH = AKCH // NC
ABK = 56
ANB = AKCH // ABK
KBUF = 14
ROWS_PER_TILE = N_PAD // NS


def _deg_body(dst_hbm, zeros_hbm, ones_hbm, pdeg_hbm, dstv, onesv, dsem, deg_sp):
  cid = lax.axis_index("c")
  sid = lax.axis_index("s")
  r0 = sid * ROWS_PER_TILE
  pltpu.sync_copy(zeros_hbm.at[pl.ds(r0, ROWS_PER_TILE)],
                  deg_sp.at[pl.ds(r0, ROWS_PER_TILE)])
  pltpu.sync_copy(dst_hbm.at[sid, pl.ds(cid * DKCH, DKCH)], dstv)
  pltpu.sync_copy(ones_hbm, onesv)
  plsc.subcore_barrier()

  def body(g, carry):
    puts = [
        pltpu.async_copy(onesv, deg_sp.at[dstv.at[g * 8 + k]], dsem.at[k],
                         add=True)
        for k in range(8)
    ]
    for p in puts:
      p.wait()
    return carry

  lax.fori_loop(0, DKCH // 8, body, 0)
  plsc.subcore_barrier()
  pltpu.sync_copy(deg_sp.at[pl.ds(r0, ROWS_PER_TILE)],
                  pdeg_hbm.at[pl.ds(cid * N_PAD + r0, ROWS_PER_TILE)])


def _agg_body(src_hbm, dst_hbm, tlo_hbm, thi_hbm, zeros_hbm, out_hbm,
              srcv, dstv, rows, gsem, ssem, agg_sp):
  cid = lax.axis_index("c")
  sid = lax.axis_index("s")
  r0 = sid * ROWS_PER_TILE
  pltpu.sync_copy(zeros_hbm.at[pl.ds(r0, ROWS_PER_TILE)],
                  agg_sp.at[pl.ds(r0, ROWS_PER_TILE)])
  plsc.subcore_barrier()

  def run_all(table_hbm):
    def blk_body(b, carry):
      pltpu.sync_copy(src_hbm.at[sid, pl.ds(b * ABK, ABK)], srcv)
      pltpu.sync_copy(dst_hbm.at[sid, pl.ds(b * ABK, ABK)], dstv)

      def group(g, c2):
        j0 = g * KBUF
        gets = [
            pltpu.async_copy(table_hbm.at[srcv.at[j0 + k]], rows.at[k],
                             gsem.at[k])
            for k in range(KBUF)
        ]
        puts = []
        for k in range(KBUF):
          gets[k].wait()
          puts.append(
              pltpu.async_copy(rows.at[k], agg_sp.at[dstv.at[j0 + k]],
                               ssem.at[k], add=True))
        for p in puts:
          p.wait()
        return c2

      lax.fori_loop(0, ABK // KBUF, group, 0)
      return carry

    lax.fori_loop(0, ANB, blk_body, 0)

  @pl.when(cid == 0)
  def _():
    run_all(tlo_hbm)

  @pl.when(cid == 1)
  def _():
    run_all(thi_hbm)

  plsc.subcore_barrier()
  pltpu.sync_copy(agg_sp.at[pl.ds(r0, ROWS_PER_TILE)],
                  out_hbm.at[cid, pl.ds(r0, ROWS_PER_TILE)])


@functools.cache
def _sc_calls():
  mesh = plsc.VectorSubcoreMesh(
      core_axis_name="c", subcore_axis_name="s",
      num_cores=NC, num_subcores=NS)
  params = pltpu.CompilerParams(use_tc_tiling_on_sc=False)
  deg_call = pl.kernel(
      _deg_body,
      out_type=jax.ShapeDtypeStruct((NC * N_PAD,), jnp.float32),
      mesh=mesh,
      compiler_params=params,
      scratch_types=[
          pltpu.VMEM((DKCH, CHUNK), jnp.int32),
          pltpu.VMEM((CHUNK,), jnp.float32),
          pltpu.SemaphoreType.DMA((8,)),
          pltpu.VMEM_SHARED((N_PAD,), jnp.float32),
      ],
  )
  agg_call = pl.kernel(
      _agg_body,
      out_type=jax.ShapeDtypeStruct((NC, N_PAD, F_HALF), jnp.float32),
      mesh=mesh,
      compiler_params=params,
      scratch_types=[
          pltpu.VMEM((ABK, CHUNK), jnp.int32),
          pltpu.VMEM((ABK, CHUNK), jnp.int32),
          pltpu.VMEM((KBUF, CHUNK, F_HALF), jnp.float32),
          pltpu.SemaphoreType.DMA((KBUF,)),
          pltpu.SemaphoreType.DMA((KBUF,)),
          pltpu.VMEM_SHARED((N_PAD, F_HALF), jnp.float32),
      ],
  )
  return deg_call, agg_call


_BLK = 2560
_GRID = N_PAD // _BLK


def _prep_body(pdeg_ref, x_ref, dis_ref, tlo_ref, thi_ref):
  deg = pdeg_ref[:, 0:1] + pdeg_ref[:, 1:2] + 1.0
  dis = lax.rsqrt(deg)
  dis_ref[...] = dis
  xn = x_ref[...] * dis
  tlo_ref[...] = xn[:, :F_HALF]
  thi_ref[...] = xn[:, F_HALF:]


_prep_call = pl.pallas_call(
    _prep_body,
    grid=(_GRID,),
    in_specs=[
        pl.BlockSpec((_BLK, NC), lambda i: (i, 0)),
        pl.BlockSpec((_BLK, F_IN), lambda i: (i, 0)),
    ],
    out_specs=[
        pl.BlockSpec((_BLK, 1), lambda i: (i, 0)),
        pl.BlockSpec((_BLK, F_HALF), lambda i: (i, 0)),
        pl.BlockSpec((_BLK, F_HALF), lambda i: (i, 0)),
    ],
    out_shape=[
        jax.ShapeDtypeStruct((N_PAD, 1), jnp.float32),
        jax.ShapeDtypeStruct((N_PAD, F_HALF), jnp.float32),
        jax.ShapeDtypeStruct((N_PAD, F_HALF), jnp.float32),
    ],
)


def _mid_body(agg_ref, dis_ref, x_ref, w1_ref, b1_ref, w2_ref,
              gn16_ref, glo_ref, ghi_ref):
  a16 = jnp.concatenate([agg_ref[0], agg_ref[1]], axis=1)
  dis = dis_ref[...]
  s = a16 * dis + x_ref[...] * (dis * dis)
  h1 = jnp.dot(s, w1_ref[...], preferred_element_type=jnp.float32)
  h1 = jnp.maximum(h1 + b1_ref[...], 0.0)
  g = jnp.dot(h1, w2_ref[...], preferred_element_type=jnp.float32)
  gn = g * dis
  gn16_ref[...] = gn
  glo_ref[...] = gn[:, :F_HALF]
  ghi_ref[...] = gn[:, F_HALF:]


_mid_call = pl.pallas_call(
    _mid_body,
    grid=(_GRID,),
    in_specs=[
        pl.BlockSpec((NC, _BLK, F_HALF), lambda i: (0, i, 0)),
        pl.BlockSpec((_BLK, 1), lambda i: (i, 0)),
        pl.BlockSpec((_BLK, F_IN), lambda i: (i, 0)),
        pl.BlockSpec((F_IN, H_MID), lambda i: (0, 0)),
        pl.BlockSpec((1, H_MID), lambda i: (0, 0)),
        pl.BlockSpec((H_MID, F_IN), lambda i: (0, 0)),
    ],
    out_specs=[
        pl.BlockSpec((_BLK, F_IN), lambda i: (i, 0)),
        pl.BlockSpec((_BLK, F_HALF), lambda i: (i, 0)),
        pl.BlockSpec((_BLK, F_HALF), lambda i: (i, 0)),
    ],
    out_shape=[
        jax.ShapeDtypeStruct((N_PAD, F_IN), jnp.float32),
        jax.ShapeDtypeStruct((N_PAD, F_HALF), jnp.float32),
        jax.ShapeDtypeStruct((N_PAD, F_HALF), jnp.float32),
    ],
)


def _fin_body(agg_ref, gn16_ref, dis_ref, b2_ref, x_ref, out_ref):
  a16 = jnp.concatenate([agg_ref[0], agg_ref[1]], axis=1)
  t = (a16 + gn16_ref[...]) * dis_ref[...] + b2_ref[...]
  out_ref[...] = 0.2 * t + 0.8 * x_ref[...]


_fin_call = pl.pallas_call(
    _fin_body,
    grid=(_GRID,),
    in_specs=[
        pl.BlockSpec((NC, _BLK, F_HALF), lambda i: (0, i, 0)),
        pl.BlockSpec((_BLK, F_IN), lambda i: (i, 0)),
        pl.BlockSpec((_BLK, 1), lambda i: (i, 0)),
        pl.BlockSpec((1, F_IN), lambda i: (0, 0)),
        pl.BlockSpec((_BLK, F_IN), lambda i: (i, 0)),
    ],
    out_specs=pl.BlockSpec((_BLK, F_IN), lambda i: (i, 0)),
    out_shape=jax.ShapeDtypeStruct((N_NODES, F_IN), jnp.float32),
)


def kernel(x, edge_index, W1, b1, W2, b2):
  ei = edge_index.astype(jnp.int32)
  pad_n = E_PAD - N_EDGES
  srcp = jnp.pad(ei[0], (0, pad_n), constant_values=N_NODES).reshape(
      NS, AKCH, CHUNK)
  dstp = jnp.pad(ei[1], (0, pad_n), constant_values=N_NODES).reshape(
      NS, AKCH, CHUNK)
  xp = jnp.pad(x, ((0, N_PAD - N_NODES), (0, 0)))
  z1 = jnp.zeros((N_PAD,), jnp.float32)
  z8 = jnp.zeros((N_PAD, F_HALF), jnp.float32)
  ones = jnp.ones((CHUNK,), jnp.float32)

  _deg_call, _agg_call = _sc_calls()
  pdeg = _deg_call(dstp, z1, ones)
  pdegT = pdeg.reshape(NC, N_PAD).T
  dis, tlo, thi = _prep_call(pdegT, xp)
  agg1 = _agg_call(srcp, dstp, tlo, thi, z8)
  gn16, glo, ghi = _mid_call(agg1, dis, xp, W1, b1.reshape(1, H_MID), W2)
  agg2 = _agg_call(srcp, dstp, glo, ghi, z8)
  return _fin_call(agg2, gn16, dis, b2.reshape(1, F_IN), xp)


# speedup vs baseline: 46.3021x; 1.2358x over previous
import functools

import jax
import jax.numpy as jnp
from jax import lax
from jax.experimental import pallas as pl
from jax.experimental.pallas import tpu as pltpu
from jax.experimental.pallas import tpu_sc as plsc

N_NODES = 100000
N_PAD = 102400
N_EDGES = 1600000
F_IN = 16
F_HALF = 8
H_MID = 32
NC, NS = 2, 16
CHUNK = 128
E_PAD = 1605632
AKCH = E_PAD // (NS * CHUNK)
DKCH = AKCH // NC
ABK = 56
ANB = AKCH // ABK
KBUF = 14
ROWS_PER_TILE = N_PAD // NS


def _deg_body(dst_hbm, zeros_hbm, ones_hbm, pdeg_hbm, dstv, onesv, dsem, deg_sp):
  cid = lax.axis_index("c")
  sid = lax.axis_index("s")
  r0 = sid * ROWS_PER_TILE
  pltpu.sync_copy(zeros_hbm.at[pl.ds(r0, ROWS_PER_TILE)],
                  deg_sp.at[pl.ds(r0, ROWS_PER_TILE)])
  pltpu.sync_copy(dst_hbm.at[sid, pl.ds(cid * DKCH, DKCH)], dstv)
  pltpu.sync_copy(ones_hbm, onesv)
  plsc.subcore_barrier()

  def body(g, carry):
    puts = [
        pltpu.async_copy(onesv, deg_sp.at[dstv.at[g * 8 + k]], dsem.at[k],
                         add=True)
        for k in range(8)
    ]
    for p in puts:
      p.wait()
    return carry

  lax.fori_loop(0, DKCH // 8, body, 0)
  plsc.subcore_barrier()
  pltpu.sync_copy(deg_sp.at[pl.ds(r0, ROWS_PER_TILE)],
                  pdeg_hbm.at[pl.ds(cid * N_PAD + r0, ROWS_PER_TILE)])


def _agg_body(src_hbm, dst_hbm, tlo_hbm, thi_hbm, out_hbm,
              srcv, dstv, rows, gsem, ssem, agg_sp):
  cid = lax.axis_index("c")
  sid = lax.axis_index("s")
  r0 = sid * ROWS_PER_TILE

  @pl.when(cid == 0)
  def _():
    pltpu.sync_copy(tlo_hbm.at[pl.ds(r0, ROWS_PER_TILE)],
                    agg_sp.at[pl.ds(r0, ROWS_PER_TILE)])

  @pl.when(cid == 1)
  def _():
    pltpu.sync_copy(thi_hbm.at[pl.ds(r0, ROWS_PER_TILE)],
                    agg_sp.at[pl.ds(r0, ROWS_PER_TILE)])

  plsc.subcore_barrier()

  def run_all(table_hbm):
    def blk_body(b, carry):
      pltpu.sync_copy(src_hbm.at[sid, pl.ds(b * ABK, ABK)], srcv)
      pltpu.sync_copy(dst_hbm.at[sid, pl.ds(b * ABK, ABK)], dstv)

      def group(g, c2):
        j0 = g * KBUF
        gets = [
            pltpu.async_copy(table_hbm.at[srcv.at[j0 + k]], rows.at[k],
                             gsem.at[k])
            for k in range(KBUF)
        ]
        puts = []
        for k in range(KBUF):
          gets[k].wait()
          puts.append(
              pltpu.async_copy(rows.at[k], agg_sp.at[dstv.at[j0 + k]],
                               ssem.at[k], add=True))
        for p in puts:
          p.wait()
        return c2

      lax.fori_loop(0, ABK // KBUF, group, 0)
      return carry

    lax.fori_loop(0, ANB, blk_body, 0)

  @pl.when(cid == 0)
  def _():
    run_all(tlo_hbm)

  @pl.when(cid == 1)
  def _():
    run_all(thi_hbm)

  plsc.subcore_barrier()
  pltpu.sync_copy(agg_sp.at[pl.ds(r0, ROWS_PER_TILE)],
                  out_hbm.at[cid, pl.ds(r0, ROWS_PER_TILE)])


@functools.cache
def _sc_calls():
  mesh = plsc.VectorSubcoreMesh(
      core_axis_name="c", subcore_axis_name="s",
      num_cores=NC, num_subcores=NS)
  params = pltpu.CompilerParams(use_tc_tiling_on_sc=False)
  deg_call = pl.kernel(
      _deg_body,
      out_type=jax.ShapeDtypeStruct((NC * N_PAD,), jnp.float32),
      mesh=mesh,
      compiler_params=params,
      scratch_types=[
          pltpu.VMEM((DKCH, CHUNK), jnp.int32),
          pltpu.VMEM((CHUNK,), jnp.float32),
          pltpu.SemaphoreType.DMA((8,)),
          pltpu.VMEM_SHARED((N_PAD,), jnp.float32),
      ],
  )
  agg_call = pl.kernel(
      _agg_body,
      out_type=jax.ShapeDtypeStruct((NC, N_PAD, F_HALF), jnp.float32),
      mesh=mesh,
      compiler_params=params,
      scratch_types=[
          pltpu.VMEM((ABK, CHUNK), jnp.int32),
          pltpu.VMEM((ABK, CHUNK), jnp.int32),
          pltpu.VMEM((KBUF, CHUNK, F_HALF), jnp.float32),
          pltpu.SemaphoreType.DMA((KBUF,)),
          pltpu.SemaphoreType.DMA((KBUF,)),
          pltpu.VMEM_SHARED((N_PAD, F_HALF), jnp.float32),
      ],
  )
  return deg_call, agg_call


_BLK = 2560
_GRID = N_PAD // _BLK


def _prep_body(pdeg_ref, x_ref, dis_ref, dis8_ref, tlo_ref, thi_ref):
  deg = pdeg_ref[:, 0:1] + pdeg_ref[:, 1:2] + 1.0
  dis = lax.rsqrt(deg)
  dis_ref[...] = dis
  dis8_ref[...] = jnp.broadcast_to(dis, (_BLK, F_HALF))
  xn = x_ref[...] * dis
  tlo_ref[...] = xn[:, :F_HALF]
  thi_ref[...] = xn[:, F_HALF:]


_prep_call = pl.pallas_call(
    _prep_body,
    grid=(_GRID,),
    in_specs=[
        pl.BlockSpec((_BLK, NC), lambda i: (i, 0)),
        pl.BlockSpec((_BLK, F_IN), lambda i: (i, 0)),
    ],
    out_specs=[
        pl.BlockSpec((_BLK, 1), lambda i: (i, 0)),
        pl.BlockSpec((_BLK, F_HALF), lambda i: (i, 0)),
        pl.BlockSpec((_BLK, F_HALF), lambda i: (i, 0)),
        pl.BlockSpec((_BLK, F_HALF), lambda i: (i, 0)),
    ],
    out_shape=[
        jax.ShapeDtypeStruct((N_PAD, 1), jnp.float32),
        jax.ShapeDtypeStruct((N_PAD, F_HALF), jnp.float32),
        jax.ShapeDtypeStruct((N_PAD, F_HALF), jnp.float32),
        jax.ShapeDtypeStruct((N_PAD, F_HALF), jnp.float32),
    ],
)


FROWS = N_PAD * F_HALF // 128
FBLK = FROWS // _GRID


def _mid_body(agg_ref, dis8_ref, w1lo_ref, w1hi_ref, b1t_ref, w2lo_ref,
              w2hi_ref, glo_ref, ghi_ref):
  dis8 = dis8_ref[...]
  s_lo = agg_ref[0] * dis8
  s_hi = agg_ref[1] * dis8
  h1 = (jnp.dot(s_lo, w1lo_ref[...], preferred_element_type=jnp.float32)
        + jnp.dot(s_hi, w1hi_ref[...], preferred_element_type=jnp.float32))
  h1 = jnp.maximum(h1 + b1t_ref[...], 0.0)
  glo_ref[...] = jnp.dot(
      h1, w2lo_ref[...], preferred_element_type=jnp.float32) * dis8
  ghi_ref[...] = jnp.dot(
      h1, w2hi_ref[...], preferred_element_type=jnp.float32) * dis8


_mid_call = pl.pallas_call(
    _mid_body,
    grid=(_GRID,),
    in_specs=[
        pl.BlockSpec((NC, FBLK, 128), lambda i: (0, i, 0)),
        pl.BlockSpec((FBLK, 128), lambda i: (i, 0)),
        pl.BlockSpec((128, 16 * H_MID), lambda i: (0, 0)),
        pl.BlockSpec((128, 16 * H_MID), lambda i: (0, 0)),
        pl.BlockSpec((1, 16 * H_MID), lambda i: (0, 0)),
        pl.BlockSpec((16 * H_MID, 128), lambda i: (0, 0)),
        pl.BlockSpec((16 * H_MID, 128), lambda i: (0, 0)),
    ],
    out_specs=[
        pl.BlockSpec((FBLK, 128), lambda i: (i, 0)),
        pl.BlockSpec((FBLK, 128), lambda i: (i, 0)),
    ],
    out_shape=[
        jax.ShapeDtypeStruct((FROWS, 128), jnp.float32),
        jax.ShapeDtypeStruct((FROWS, 128), jnp.float32),
    ],
)


def _fin_body(agg_ref, dis_ref, b2_ref, x_ref, out_ref):
  a16 = jnp.concatenate([agg_ref[0], agg_ref[1]], axis=1)
  t = a16 * dis_ref[...] + b2_ref[...]
  out_ref[...] = 0.2 * t + 0.8 * x_ref[...]


_fin_call = pl.pallas_call(
    _fin_body,
    grid=(_GRID,),
    in_specs=[
        pl.BlockSpec((NC, _BLK, F_HALF), lambda i: (0, i, 0)),
        pl.BlockSpec((_BLK, 1), lambda i: (i, 0)),
        pl.BlockSpec((1, F_IN), lambda i: (0, 0)),
        pl.BlockSpec((_BLK, F_IN), lambda i: (i, 0)),
    ],
    out_specs=pl.BlockSpec((_BLK, F_IN), lambda i: (i, 0)),
    out_shape=jax.ShapeDtypeStruct((N_NODES, F_IN), jnp.float32),
)


def kernel(x, edge_index, W1, b1, W2, b2):
  ei = edge_index.astype(jnp.int32)
  pad_n = E_PAD - N_EDGES
  srcp = jnp.pad(ei[0], (0, pad_n), constant_values=N_NODES).reshape(
      NS, AKCH, CHUNK)
  dstp = jnp.pad(ei[1], (0, pad_n), constant_values=N_NODES).reshape(
      NS, AKCH, CHUNK)
  xp = jnp.pad(x, ((0, N_PAD - N_NODES), (0, 0)))
  z1 = jnp.zeros((N_PAD,), jnp.float32)
  ones = jnp.ones((CHUNK,), jnp.float32)
  eye16 = jnp.eye(16, dtype=jnp.float32)
  w1lo = jnp.kron(eye16, W1[:F_HALF, :])
  w1hi = jnp.kron(eye16, W1[F_HALF:, :])
  w2lo = jnp.kron(eye16, W2[:, :F_HALF])
  w2hi = jnp.kron(eye16, W2[:, F_HALF:])
  b1t = jnp.tile(b1, 16).reshape(1, 16 * H_MID)

  _deg_call, _agg_call = _sc_calls()
  pdeg = _deg_call(dstp, z1, ones)
  pdegT = pdeg.reshape(NC, N_PAD).T
  dis, dis8, tlo, thi = _prep_call(pdegT, xp)
  agg1 = _agg_call(srcp, dstp, tlo, thi)
  agg1v = agg1.reshape(NC, FROWS, 128)
  dis8f = dis8.reshape(FROWS, 128)
  glo, ghi = _mid_call(agg1v, dis8f, w1lo, w1hi, b1t, w2lo, w2hi)
  agg2 = _agg_call(srcp, dstp,
                   glo.reshape(N_PAD, F_HALF), ghi.reshape(N_PAD, F_HALF))
  return _fin_call(agg2, dis, b2.reshape(1, F_IN), xp)


# speedup vs baseline: 47.4382x; 1.0245x over previous
import functools

import jax
import jax.numpy as jnp
from jax import lax
from jax.experimental import pallas as pl
from jax.experimental.pallas import tpu as pltpu
from jax.experimental.pallas import tpu_sc as plsc

N_NODES = 100000
N_PAD = 102400
N_EDGES = 1600000
F_IN = 16
F_HALF = 8
H_MID = 32
NC, NS = 2, 16
CHUNK = 125
ECH = N_EDGES // CHUNK
AKCH = ECH // NS
DKCH = ECH // (NS * NC)
ABK = 50
ANB = AKCH // ABK
KBUF = 10
ROWS_PER_TILE = N_PAD // NS


def _deg_body(edge_hbm, zeros_hbm, ones_hbm, pdeg_hbm, dstv, onesv, dsem, deg_sp):
  cid = lax.axis_index("c")
  sid = lax.axis_index("s")
  wid = sid * NC + cid
  r0 = sid * ROWS_PER_TILE
  pltpu.sync_copy(zeros_hbm.at[pl.ds(r0, ROWS_PER_TILE)],
                  deg_sp.at[pl.ds(r0, ROWS_PER_TILE)])
  pltpu.sync_copy(edge_hbm.at[1, pl.ds(wid * DKCH, DKCH)], dstv)
  pltpu.sync_copy(ones_hbm, onesv)
  plsc.subcore_barrier()

  def body(g, carry):
    puts = [
        pltpu.async_copy(onesv, deg_sp.at[dstv.at[g * 8 + k]], dsem.at[k],
                         add=True)
        for k in range(8)
    ]
    for p in puts:
      p.wait()
    return carry

  lax.fori_loop(0, DKCH // 8, body, 0)
  plsc.subcore_barrier()
  pltpu.sync_copy(deg_sp.at[pl.ds(r0, ROWS_PER_TILE)],
                  pdeg_hbm.at[pl.ds(cid * N_PAD + r0, ROWS_PER_TILE)])


def _agg_body(edge_hbm, tlo_hbm, thi_hbm, out_hbm,
              srcv, dstv, rows, gsem, ssem, agg_sp):
  cid = lax.axis_index("c")
  sid = lax.axis_index("s")
  r0 = sid * ROWS_PER_TILE

  @pl.when(cid == 0)
  def _():
    pltpu.sync_copy(tlo_hbm.at[pl.ds(r0, ROWS_PER_TILE)],
                    agg_sp.at[pl.ds(r0, ROWS_PER_TILE)])

  @pl.when(cid == 1)
  def _():
    pltpu.sync_copy(thi_hbm.at[pl.ds(r0, ROWS_PER_TILE)],
                    agg_sp.at[pl.ds(r0, ROWS_PER_TILE)])

  plsc.subcore_barrier()

  def run_all(table_hbm):
    def blk_body(b, carry):
      c0 = sid * AKCH + b * ABK
      pltpu.sync_copy(edge_hbm.at[0, pl.ds(c0, ABK)], srcv)
      pltpu.sync_copy(edge_hbm.at[1, pl.ds(c0, ABK)], dstv)

      def group(g, c2):
        j0 = g * KBUF
        gets = [
            pltpu.async_copy(table_hbm.at[srcv.at[j0 + k]], rows.at[k],
                             gsem.at[k])
            for k in range(KBUF)
        ]
        puts = []
        for k in range(KBUF):
          gets[k].wait()
          puts.append(
              pltpu.async_copy(rows.at[k], agg_sp.at[dstv.at[j0 + k]],
                               ssem.at[k], add=True))
        for p in puts:
          p.wait()
        return c2

      lax.fori_loop(0, ABK // KBUF, group, 0)
      return carry

    lax.fori_loop(0, ANB, blk_body, 0)

  @pl.when(cid == 0)
  def _():
    run_all(tlo_hbm)

  @pl.when(cid == 1)
  def _():
    run_all(thi_hbm)

  plsc.subcore_barrier()
  pltpu.sync_copy(agg_sp.at[pl.ds(r0, ROWS_PER_TILE)],
                  out_hbm.at[cid, pl.ds(r0, ROWS_PER_TILE)])


@functools.cache
def _sc_calls():
  mesh = plsc.VectorSubcoreMesh(
      core_axis_name="c", subcore_axis_name="s",
      num_cores=NC, num_subcores=NS)
  params = pltpu.CompilerParams(use_tc_tiling_on_sc=False)
  deg_call = pl.kernel(
      _deg_body,
      out_type=jax.ShapeDtypeStruct((NC * N_PAD,), jnp.float32),
      mesh=mesh,
      compiler_params=params,
      scratch_types=[
          pltpu.VMEM((DKCH, CHUNK), jnp.int32),
          pltpu.VMEM((CHUNK,), jnp.float32),
          pltpu.SemaphoreType.DMA((8,)),
          pltpu.VMEM_SHARED((N_PAD,), jnp.float32),
      ],
  )
  agg_call = pl.kernel(
      _agg_body,
      out_type=jax.ShapeDtypeStruct((NC, N_PAD, F_HALF), jnp.float32),
      mesh=mesh,
      compiler_params=params,
      scratch_types=[
          pltpu.VMEM((ABK, CHUNK), jnp.int32),
          pltpu.VMEM((ABK, CHUNK), jnp.int32),
          pltpu.VMEM((KBUF, CHUNK, F_HALF), jnp.float32),
          pltpu.SemaphoreType.DMA((KBUF,)),
          pltpu.SemaphoreType.DMA((KBUF,)),
          pltpu.VMEM_SHARED((N_PAD, F_HALF), jnp.float32),
      ],
  )
  return deg_call, agg_call


_BLK = 2560
_GRID = N_PAD // _BLK


def _prep_body(pdeg_ref, x_ref, dis8_ref, tlo_ref, thi_ref):
  deg = pdeg_ref[:, 0:1] + pdeg_ref[:, 1:2] + 1.0
  dis = lax.rsqrt(deg)
  dis8_ref[...] = jnp.broadcast_to(dis, (_BLK, F_HALF))
  xn = x_ref[...] * dis
  tlo_ref[...] = xn[:, :F_HALF]
  thi_ref[...] = xn[:, F_HALF:]


_prep_call = pl.pallas_call(
    _prep_body,
    grid=(_GRID,),
    in_specs=[
        pl.BlockSpec((_BLK, NC), lambda i: (i, 0)),
        pl.BlockSpec((_BLK, F_IN), lambda i: (i, 0)),
    ],
    out_specs=[
        pl.BlockSpec((_BLK, F_HALF), lambda i: (i, 0)),
        pl.BlockSpec((_BLK, F_HALF), lambda i: (i, 0)),
        pl.BlockSpec((_BLK, F_HALF), lambda i: (i, 0)),
    ],
    out_shape=[
        jax.ShapeDtypeStruct((N_PAD, F_HALF), jnp.float32),
        jax.ShapeDtypeStruct((N_PAD, F_HALF), jnp.float32),
        jax.ShapeDtypeStruct((N_PAD, F_HALF), jnp.float32),
    ],
)


FROWS = N_PAD * F_HALF // 128
FBLK = FROWS // _GRID


def _mid_body(agg_ref, dis8_ref, w1lo_ref, w1hi_ref, b1t_ref, w2lo_ref,
              w2hi_ref, glo_ref, ghi_ref):
  dis8 = dis8_ref[...]
  s_lo = agg_ref[0] * dis8
  s_hi = agg_ref[1] * dis8
  h1 = (jnp.dot(s_lo, w1lo_ref[...], preferred_element_type=jnp.float32)
        + jnp.dot(s_hi, w1hi_ref[...], preferred_element_type=jnp.float32))
  h1 = jnp.maximum(h1 + b1t_ref[...], 0.0)
  glo_ref[...] = jnp.dot(
      h1, w2lo_ref[...], preferred_element_type=jnp.float32) * dis8
  ghi_ref[...] = jnp.dot(
      h1, w2hi_ref[...], preferred_element_type=jnp.float32) * dis8


_mid_call = pl.pallas_call(
    _mid_body,
    grid=(_GRID,),
    in_specs=[
        pl.BlockSpec((NC, FBLK, 128), lambda i: (0, i, 0)),
        pl.BlockSpec((FBLK, 128), lambda i: (i, 0)),
        pl.BlockSpec((128, 16 * H_MID), lambda i: (0, 0)),
        pl.BlockSpec((128, 16 * H_MID), lambda i: (0, 0)),
        pl.BlockSpec((1, 16 * H_MID), lambda i: (0, 0)),
        pl.BlockSpec((16 * H_MID, 128), lambda i: (0, 0)),
        pl.BlockSpec((16 * H_MID, 128), lambda i: (0, 0)),
    ],
    out_specs=[
        pl.BlockSpec((FBLK, 128), lambda i: (i, 0)),
        pl.BlockSpec((FBLK, 128), lambda i: (i, 0)),
    ],
    out_shape=[
        jax.ShapeDtypeStruct((FROWS, 128), jnp.float32),
        jax.ShapeDtypeStruct((FROWS, 128), jnp.float32),
    ],
)


def _fin_body(agg_ref, dis8_ref, b2_ref, x_ref, out_ref):
  dis8 = dis8_ref[...]
  t = jnp.concatenate([agg_ref[0] * dis8, agg_ref[1] * dis8], axis=1)
  t = t + b2_ref[...]
  out_ref[...] = 0.2 * t + 0.8 * x_ref[...]


_fin_call = pl.pallas_call(
    _fin_body,
    grid=(_GRID,),
    in_specs=[
        pl.BlockSpec((NC, _BLK, F_HALF), lambda i: (0, i, 0)),
        pl.BlockSpec((_BLK, F_HALF), lambda i: (i, 0)),
        pl.BlockSpec((1, F_IN), lambda i: (0, 0)),
        pl.BlockSpec((_BLK, F_IN), lambda i: (i, 0)),
    ],
    out_specs=pl.BlockSpec((_BLK, F_IN), lambda i: (i, 0)),
    out_shape=jax.ShapeDtypeStruct((N_NODES, F_IN), jnp.float32),
)


def kernel(x, edge_index, W1, b1, W2, b2):
  e3 = edge_index.astype(jnp.int32).reshape(2, ECH, CHUNK)
  xp = jnp.pad(x, ((0, N_PAD - N_NODES), (0, 0)))
  z1 = jnp.zeros((N_PAD,), jnp.float32)
  ones = jnp.ones((CHUNK,), jnp.float32)
  eye16 = jnp.eye(16, dtype=jnp.float32)
  w1lo = jnp.kron(eye16, W1[:F_HALF, :])
  w1hi = jnp.kron(eye16, W1[F_HALF:, :])
  w2lo = jnp.kron(eye16, W2[:, :F_HALF])
  w2hi = jnp.kron(eye16, W2[:, F_HALF:])
  b1t = jnp.tile(b1, 16).reshape(1, 16 * H_MID)

  _deg_call, _agg_call = _sc_calls()
  pdeg = _deg_call(e3, z1, ones)
  pdegT = pdeg.reshape(NC, N_PAD).T
  dis8, tlo, thi = _prep_call(pdegT, xp)
  agg1 = _agg_call(e3, tlo, thi)
  agg1v = agg1.reshape(NC, FROWS, 128)
  dis8f = dis8.reshape(FROWS, 128)
  glo, ghi = _mid_call(agg1v, dis8f, w1lo, w1hi, b1t, w2lo, w2hi)
  agg2 = _agg_call(e3, glo.reshape(N_PAD, F_HALF), ghi.reshape(N_PAD, F_HALF))
  return _fin_call(agg2, dis8, b2.reshape(1, F_IN), xp)


# speedup vs baseline: 48.2693x; 1.0175x over previous
import functools

import jax
import jax.numpy as jnp
from jax import lax
from jax.experimental import pallas as pl
from jax.experimental.pallas import tpu as pltpu
from jax.experimental.pallas import tpu_sc as plsc

N_NODES = 100000
N_PAD = 102400
N_EDGES = 1600000
F_IN = 16
F_HALF = 8
H_MID = 32
NC, NS = 2, 16
CHUNK = 125
ECH = N_EDGES // CHUNK
AKCH = ECH // NS
DKCH = ECH // (NS * NC)
ABK = 50
ANB = AKCH // ABK
KBUF = 10
ROWS_PER_TILE = N_PAD // NS


def _deg_body(edge_hbm, zeros_hbm, ones_hbm, pdeg_hbm, dstv, onesv, dsem, deg_sp):
  cid = lax.axis_index("c")
  sid = lax.axis_index("s")
  wid = sid * NC + cid
  r0 = sid * ROWS_PER_TILE
  pltpu.sync_copy(zeros_hbm.at[pl.ds(r0, ROWS_PER_TILE)],
                  deg_sp.at[pl.ds(r0, ROWS_PER_TILE)])
  pltpu.sync_copy(edge_hbm.at[1, pl.ds(wid * DKCH, DKCH)], dstv)
  pltpu.sync_copy(ones_hbm, onesv)
  plsc.subcore_barrier()

  def body(g, carry):
    puts = [
        pltpu.async_copy(onesv, deg_sp.at[dstv.at[g * 8 + k]], dsem.at[k],
                         add=True)
        for k in range(8)
    ]
    for p in puts:
      p.wait()
    return carry

  lax.fori_loop(0, DKCH // 8, body, 0)
  plsc.subcore_barrier()
  pltpu.sync_copy(deg_sp.at[pl.ds(r0, ROWS_PER_TILE)],
                  pdeg_hbm.at[pl.ds(cid * N_PAD + r0, ROWS_PER_TILE)])


def _agg_body(edge_hbm, tlo_hbm, thi_hbm, out_hbm,
              srcv, dstv, rows, gsem, ssem, agg_sp):
  cid = lax.axis_index("c")
  sid = lax.axis_index("s")
  r0 = sid * ROWS_PER_TILE

  @pl.when(cid == 0)
  def _():
    pltpu.sync_copy(tlo_hbm.at[pl.ds(r0, ROWS_PER_TILE)],
                    agg_sp.at[pl.ds(r0, ROWS_PER_TILE)])

  @pl.when(cid == 1)
  def _():
    pltpu.sync_copy(thi_hbm.at[pl.ds(r0, ROWS_PER_TILE)],
                    agg_sp.at[pl.ds(r0, ROWS_PER_TILE)])

  plsc.subcore_barrier()

  def run_all(table_hbm):
    def blk_body(b, carry):
      c0 = sid * AKCH + b * ABK
      pltpu.sync_copy(edge_hbm.at[0, pl.ds(c0, ABK)], srcv)
      pltpu.sync_copy(edge_hbm.at[1, pl.ds(c0, ABK)], dstv)

      def group(g, c2):
        j0 = g * KBUF
        gets = [
            pltpu.async_copy(table_hbm.at[srcv.at[j0 + k]], rows.at[k],
                             gsem.at[k])
            for k in range(KBUF)
        ]
        puts = []
        for k in range(KBUF):
          gets[k].wait()
          puts.append(
              pltpu.async_copy(rows.at[k], agg_sp.at[dstv.at[j0 + k]],
                               ssem.at[k], add=True))
        for p in puts:
          p.wait()
        return c2

      lax.fori_loop(0, ABK // KBUF, group, 0)
      return carry

    lax.fori_loop(0, ANB, blk_body, 0)

  @pl.when(cid == 0)
  def _():
    run_all(tlo_hbm)

  @pl.when(cid == 1)
  def _():
    run_all(thi_hbm)

  plsc.subcore_barrier()
  pltpu.sync_copy(agg_sp.at[pl.ds(r0, ROWS_PER_TILE)],
                  out_hbm.at[cid, pl.ds(r0, ROWS_PER_TILE)])


@functools.cache
def _sc_calls():
  mesh = plsc.VectorSubcoreMesh(
      core_axis_name="c", subcore_axis_name="s",
      num_cores=NC, num_subcores=NS)
  params = pltpu.CompilerParams(use_tc_tiling_on_sc=False)
  deg_call = pl.kernel(
      _deg_body,
      out_type=jax.ShapeDtypeStruct((NC * N_PAD,), jnp.float32),
      mesh=mesh,
      compiler_params=params,
      scratch_types=[
          pltpu.VMEM((DKCH, CHUNK), jnp.int32),
          pltpu.VMEM((CHUNK,), jnp.float32),
          pltpu.SemaphoreType.DMA((8,)),
          pltpu.VMEM_SHARED((N_PAD,), jnp.float32),
      ],
  )
  agg_call = pl.kernel(
      _agg_body,
      out_type=jax.ShapeDtypeStruct((NC, N_PAD, F_HALF), jnp.float32),
      mesh=mesh,
      compiler_params=params,
      scratch_types=[
          pltpu.VMEM((ABK, CHUNK), jnp.int32),
          pltpu.VMEM((ABK, CHUNK), jnp.int32),
          pltpu.VMEM((KBUF, CHUNK, F_HALF), jnp.float32),
          pltpu.SemaphoreType.DMA((KBUF,)),
          pltpu.SemaphoreType.DMA((KBUF,)),
          pltpu.VMEM_SHARED((N_PAD, F_HALF), jnp.float32),
      ],
  )
  return deg_call, agg_call


_BLK = 5120
_GRID = N_PAD // _BLK


def _prep_body(pdeg_ref, x_ref, dis8_ref, tlo_ref, thi_ref):
  deg = pdeg_ref[:, 0:1] + pdeg_ref[:, 1:2] + 1.0
  dis = lax.rsqrt(deg)
  dis8_ref[...] = jnp.broadcast_to(dis, (_BLK, F_HALF))
  xn = x_ref[...] * dis
  tlo_ref[...] = xn[:, :F_HALF]
  thi_ref[...] = xn[:, F_HALF:]


_prep_call = pl.pallas_call(
    _prep_body,
    grid=(_GRID,),
    in_specs=[
        pl.BlockSpec((_BLK, NC), lambda i: (i, 0)),
        pl.BlockSpec((_BLK, F_IN), lambda i: (i, 0)),
    ],
    out_specs=[
        pl.BlockSpec((_BLK, F_HALF), lambda i: (i, 0)),
        pl.BlockSpec((_BLK, F_HALF), lambda i: (i, 0)),
        pl.BlockSpec((_BLK, F_HALF), lambda i: (i, 0)),
    ],
    out_shape=[
        jax.ShapeDtypeStruct((N_PAD, F_HALF), jnp.float32),
        jax.ShapeDtypeStruct((N_PAD, F_HALF), jnp.float32),
        jax.ShapeDtypeStruct((N_PAD, F_HALF), jnp.float32),
    ],
)


FROWS = N_PAD * F_HALF // 128
FBLK = FROWS // _GRID


def _mid_body(alo_ref, ahi_ref, dis8_ref, w1lo_ref, w1hi_ref, b1t_ref,
              w2lo_ref, w2hi_ref, glo_ref, ghi_ref):
  dis8 = dis8_ref[...]
  s_lo = alo_ref[...] * dis8
  s_hi = ahi_ref[...] * dis8
  h1 = (jnp.dot(s_lo, w1lo_ref[...], preferred_element_type=jnp.float32)
        + jnp.dot(s_hi, w1hi_ref[...], preferred_element_type=jnp.float32))
  h1 = jnp.maximum(h1 + b1t_ref[...], 0.0)
  glo_ref[...] = jnp.dot(
      h1, w2lo_ref[...], preferred_element_type=jnp.float32) * dis8
  ghi_ref[...] = jnp.dot(
      h1, w2hi_ref[...], preferred_element_type=jnp.float32) * dis8


_mid_call = pl.pallas_call(
    _mid_body,
    grid=(_GRID,),
    in_specs=[
        pl.BlockSpec((FBLK, 128), lambda i: (i, 0)),
        pl.BlockSpec((FBLK, 128), lambda i: (i + _GRID, 0)),
        pl.BlockSpec((FBLK, 128), lambda i: (i, 0)),
        pl.BlockSpec((128, 16 * H_MID), lambda i: (0, 0)),
        pl.BlockSpec((128, 16 * H_MID), lambda i: (0, 0)),
        pl.BlockSpec((1, 16 * H_MID), lambda i: (0, 0)),
        pl.BlockSpec((16 * H_MID, 128), lambda i: (0, 0)),
        pl.BlockSpec((16 * H_MID, 128), lambda i: (0, 0)),
    ],
    out_specs=[
        pl.BlockSpec((FBLK, 128), lambda i: (i, 0)),
        pl.BlockSpec((FBLK, 128), lambda i: (i, 0)),
    ],
    out_shape=[
        jax.ShapeDtypeStruct((FROWS, 128), jnp.float32),
        jax.ShapeDtypeStruct((FROWS, 128), jnp.float32),
    ],
)


def _fin_body(agg_ref, dis8_ref, b2_ref, x_ref, out_ref):
  dis8 = dis8_ref[...]
  t = jnp.concatenate([agg_ref[0] * dis8, agg_ref[1] * dis8], axis=1)
  t = t + b2_ref[...]
  out_ref[...] = 0.2 * t + 0.8 * x_ref[...]


_fin_call = pl.pallas_call(
    _fin_body,
    grid=(_GRID,),
    in_specs=[
        pl.BlockSpec((NC, _BLK, F_HALF), lambda i: (0, i, 0)),
        pl.BlockSpec((_BLK, F_HALF), lambda i: (i, 0)),
        pl.BlockSpec((1, F_IN), lambda i: (0, 0)),
        pl.BlockSpec((_BLK, F_IN), lambda i: (i, 0)),
    ],
    out_specs=pl.BlockSpec((_BLK, F_IN), lambda i: (i, 0)),
    out_shape=jax.ShapeDtypeStruct((N_NODES, F_IN), jnp.float32),
)


def kernel(x, edge_index, W1, b1, W2, b2):
  e3 = edge_index.astype(jnp.int32).reshape(2, ECH, CHUNK)
  xp = jnp.pad(x, ((0, N_PAD - N_NODES), (0, 0)))
  z1 = jnp.zeros((N_PAD,), jnp.float32)
  ones = jnp.ones((CHUNK,), jnp.float32)
  eye16 = jnp.eye(16, dtype=jnp.float32)
  w1lo = jnp.kron(eye16, W1[:F_HALF, :])
  w1hi = jnp.kron(eye16, W1[F_HALF:, :])
  w2lo = jnp.kron(eye16, W2[:, :F_HALF])
  w2hi = jnp.kron(eye16, W2[:, F_HALF:])
  b1t = jnp.tile(b1, 16).reshape(1, 16 * H_MID)

  _deg_call, _agg_call = _sc_calls()
  pdeg = _deg_call(e3, z1, ones)
  pdegT = pdeg.reshape(NC, N_PAD).T
  dis8, tlo, thi = _prep_call(pdegT, xp)
  agg1 = _agg_call(e3, tlo, thi)
  agg1v = agg1.reshape(NC * FROWS, 128)
  dis8f = dis8.reshape(FROWS, 128)
  glo, ghi = _mid_call(agg1v, agg1v, dis8f, w1lo, w1hi, b1t, w2lo, w2hi)
  agg2 = _agg_call(e3, glo.reshape(N_PAD, F_HALF), ghi.reshape(N_PAD, F_HALF))
  return _fin_call(agg2, dis8, b2.reshape(1, F_IN), xp)
